# Initial kernel scaffold; baseline (speedup 1.0000x reference)
import jax
import jax.numpy as jnp
from jax.experimental import pallas as pl


def kernel(feature_idx, feature_values, table):
    raise NotImplementedError("write your pallas kernel here")


# speedup vs baseline: 1.3273x; 1.3273x over previous
import functools

import jax
import jax.numpy as jnp
from jax import lax
from jax.experimental import pallas as pl
from jax.experimental.pallas import tpu as pltpu
from jax.experimental.pallas import tpu_sc as plsc

B = 16384
F = 26
NC = 2
NS = 16
NW = NC * NS
BPW = B // NW
CHUNK = 128
NCH = (F * BPW) // CHUNK
FIRE = 8

_mesh = plsc.VectorSubcoreMesh(core_axis_name="c", subcore_axis_name="s")


@functools.partial(
    pl.kernel,
    out_type=jax.ShapeDtypeStruct((B,), jnp.float32),
    mesh=_mesh,
    scratch_types=[
        pltpu.VMEM((F, BPW), jnp.int32),
        pltpu.VMEM((F, BPW), jnp.float32),
        pltpu.VMEM((F, BPW), jnp.float32),
        pltpu.VMEM((BPW,), jnp.float32),
        pltpu.SemaphoreType.DMA,
    ],
)
def _sc_kernel(idx_hbm, vals_hbm, table_hbm, out_hbm,
               idx_v, vals_v, g_v, acc_v, sem):
    wid = lax.axis_index("s") * NC + lax.axis_index("c")
    base = wid * BPW

    pltpu.sync_copy(idx_hbm.at[wid], idx_v)
    pltpu.sync_copy(vals_hbm.at[wid], vals_v)

    def _chunk_refs(c):
        f, off = divmod(c, BPW // CHUNK)
        return (idx_v.at[f, pl.ds(off * CHUNK, CHUNK)],
                g_v.at[f, pl.ds(off * CHUNK, CHUNK)])

    for g0 in range(0, NCH, FIRE):
        copies = []
        for c in range(g0, min(g0 + FIRE, NCH)):
            isl, gsl = _chunk_refs(c)
            copies.append(pltpu.async_copy(table_hbm.at[isl], gsl, sem))
        for cp in copies:
            cp.wait()

    def _body(i, carry):
        sl = pl.ds(i * 16, 16)
        acc = g_v[0, sl] * vals_v[0, sl]
        for f in range(1, F):
            acc = acc + g_v[f, sl] * vals_v[f, sl]
        acc_v[sl] = acc
        return carry

    lax.fori_loop(0, BPW // 16, _body, 0)

    pltpu.sync_copy(acc_v, out_hbm.at[pl.ds(base, BPW)])


def kernel(feature_idx, feature_values, table):
    idx_t = feature_idx.astype(jnp.int32).T
    vals_t = feature_values.T
    idx_r = idx_t.reshape(F, NW, BPW).transpose(1, 0, 2)
    vals_r = vals_t.reshape(F, NW, BPW).transpose(1, 0, 2)
    return _sc_kernel(idx_r, vals_r, table.reshape(-1))


# speedup vs baseline: 1.4832x; 1.1174x over previous
import functools

import jax
import jax.numpy as jnp
from jax import lax
from jax.experimental import pallas as pl
from jax.experimental.pallas import tpu as pltpu
from jax.experimental.pallas import tpu_sc as plsc

B = 16384
F = 26
NC = 2
NS = 16
NW = NC * NS
BPW = B // NW
CHUNK = 128
NCH = (F * BPW) // CHUNK
FIRE = 8

_mesh = plsc.VectorSubcoreMesh(core_axis_name="c", subcore_axis_name="s")


@functools.partial(
    pl.kernel,
    out_type=jax.ShapeDtypeStruct((B,), jnp.float32),
    mesh=_mesh,
    scratch_types=[
        pltpu.VMEM((F * BPW,), jnp.int32),
        pltpu.VMEM((F * BPW,), jnp.float32),
        pltpu.VMEM((F * BPW,), jnp.float32),
        pltpu.VMEM((BPW,), jnp.float32),
        pltpu.SemaphoreType.DMA,
    ],
)
def _sc_kernel(idx_hbm, vals_hbm, table_hbm, out_hbm,
               idx_v, vals_v, g_v, acc_v, sem):
    wid = lax.axis_index("s") * NC + lax.axis_index("c")
    base = wid * BPW

    pltpu.sync_copy(idx_hbm.at[wid], idx_v)
    gcp = pltpu.async_copy(table_hbm.at[idx_v], g_v, sem)
    pltpu.sync_copy(vals_hbm.at[wid], vals_v)
    gcp.wait()

    def _body(i, carry):
        acc = g_v[pl.ds(i * 16, 16)] * vals_v[pl.ds(i * 16, 16)]
        for f in range(1, F):
            sl = pl.ds(f * BPW + i * 16, 16)
            acc = acc + g_v[sl] * vals_v[sl]
        acc_v[pl.ds(i * 16, 16)] = acc
        return carry

    lax.fori_loop(0, BPW // 16, _body, 0)

    pltpu.sync_copy(acc_v, out_hbm.at[pl.ds(base, BPW)])


def kernel(feature_idx, feature_values, table):
    idx_t = feature_idx.astype(jnp.int32).T
    vals_t = feature_values.T
    idx_r = idx_t.reshape(F, NW, BPW).transpose(1, 0, 2).reshape(NW, F * BPW)
    vals_r = vals_t.reshape(F, NW, BPW).transpose(1, 0, 2).reshape(NW, F * BPW)
    return _sc_kernel(idx_r, vals_r, table.reshape(-1))


# speedup vs baseline: 1.5759x; 1.0625x over previous
import functools

import jax
import jax.numpy as jnp
from jax import lax
from jax.experimental import pallas as pl
from jax.experimental.pallas import tpu as pltpu
from jax.experimental.pallas import tpu_sc as plsc

B = 16384
F = 26
NC = 2
NS = 16
NW = NC * NS
BPW = B // NW
N_PER_W = F * BPW

_mesh = plsc.VectorSubcoreMesh(core_axis_name="c", subcore_axis_name="s")


@functools.partial(
    pl.kernel,
    out_type=jax.ShapeDtypeStruct((B,), jnp.float32),
    mesh=_mesh,
    scratch_types=[
        pltpu.VMEM((F, BPW // 128, 128), jnp.int32),
        pltpu.VMEM((F, BPW), jnp.float32),
        pltpu.VMEM((N_PER_W,), jnp.float32),
        pltpu.VMEM((BPW,), jnp.float32),
        pltpu.SemaphoreType.DMA,
    ],
)
def _sc_kernel(idx_hbm, vals_hbm, table_hbm, out_hbm,
               idx_v, vals_v, g_v, acc_v, sem):
    wid = lax.axis_index("s") * NC + lax.axis_index("c")
    base = wid * BPW

    pltpu.sync_copy(idx_hbm.at[:, pl.ds(wid * (BPW // 128), BPW // 128), :], idx_v)
    copies = [
        pltpu.async_copy(
            table_hbm.at[idx_v.at[f, c]],
            g_v.at[pl.ds(f * BPW + c * 128, 128)], sem
        )
        for f in range(F)
        for c in range(BPW // 128)
    ]
    pltpu.sync_copy(vals_hbm.at[:, pl.ds(base, BPW)], vals_v)
    for cp in copies:
        cp.wait()

    def _body(i, carry):
        acc = g_v[pl.ds(i * 16, 16)] * vals_v[0, pl.ds(i * 16, 16)]
        for f in range(1, F):
            acc = acc + g_v[pl.ds(f * BPW + i * 16, 16)] * vals_v[f, pl.ds(i * 16, 16)]
        acc_v[pl.ds(i * 16, 16)] = acc
        return carry

    lax.fori_loop(0, BPW // 16, _body, 0)

    pltpu.sync_copy(acc_v, out_hbm.at[pl.ds(base, BPW)])


def kernel(feature_idx, feature_values, table):
    idx_t = feature_idx.astype(jnp.int32).T
    vals_t = feature_values.T
    idx_3d = idx_t.reshape(F, NW * (BPW // 128), 128)
    return _sc_kernel(idx_3d, vals_t, table.reshape(-1))


# speedup vs baseline: 1.6234x; 1.0301x over previous
import functools

import jax
import jax.numpy as jnp
from jax import lax
from jax.experimental import pallas as pl
from jax.experimental.pallas import tpu as pltpu
from jax.experimental.pallas import tpu_sc as plsc

B = 16384
F = 26
FH = F // 2
NC = 2
NS = 16
NW = NC * NS
BPW = B // NW
NH = FH * BPW

_mesh = plsc.VectorSubcoreMesh(core_axis_name="c", subcore_axis_name="s")


@functools.partial(
    pl.kernel,
    out_type=jax.ShapeDtypeStruct((B,), jnp.float32),
    mesh=_mesh,
    scratch_types=[
        pltpu.VMEM((NH,), jnp.int32),
        pltpu.VMEM((NH,), jnp.int32),
        pltpu.VMEM((F, BPW), jnp.float32),
        pltpu.VMEM((NH,), jnp.float32),
        pltpu.VMEM((NH,), jnp.float32),
        pltpu.VMEM((BPW,), jnp.float32),
        pltpu.SemaphoreType.DMA,
        pltpu.SemaphoreType.DMA,
        pltpu.SemaphoreType.DMA,
    ],
)
def _sc_kernel(idx_hbm, vals_hbm, table_hbm, out_hbm,
               idxa_v, idxb_v, vals_v, ga_v, gb_v, acc_v,
               sem_s, sem_a, sem_b):
    wid = lax.axis_index("s") * NC + lax.axis_index("c")
    base = wid * BPW

    stage = [
        pltpu.async_copy(
            idx_hbm.at[f, pl.ds(base, BPW)],
            (idxa_v if f < FH else idxb_v).at[pl.ds((f % FH) * BPW, BPW)],
            sem_s,
        )
        for f in range(F)
    ]
    for cp in stage[:FH]:
        cp.wait()
    cpa = pltpu.async_copy(table_hbm.at[idxa_v], ga_v, sem_a)
    for cp in stage[FH:]:
        cp.wait()
    cpb = pltpu.async_copy(table_hbm.at[idxb_v], gb_v, sem_b)
    pltpu.sync_copy(vals_hbm.at[:, pl.ds(base, BPW)], vals_v)

    def _half(g_v, f0, first):
        def _body(i, carry):
            sl16 = pl.ds(i * 16, 16)
            acc = jnp.zeros((16,), jnp.float32) if first else acc_v[sl16]
            for fh in range(FH):
                acc = acc + (g_v[pl.ds(fh * BPW + i * 16, 16)]
                             * vals_v[f0 + fh, sl16])
            acc_v[sl16] = acc
            return carry

        lax.fori_loop(0, BPW // 16, _body, 0)

    cpa.wait()
    _half(ga_v, 0, True)
    cpb.wait()
    _half(gb_v, FH, False)

    pltpu.sync_copy(acc_v, out_hbm.at[pl.ds(base, BPW)])


def kernel(feature_idx, feature_values, table):
    idx_t = feature_idx.astype(jnp.int32).T
    vals_t = feature_values.T
    return _sc_kernel(idx_t, vals_t, table.reshape(-1))
